# baseline (speedup 1.0000x reference)
import functools

import jax
import jax.numpy as jnp
from jax import lax
from jax.experimental import pallas as pl
from jax.experimental.pallas import tpu as pltpu
from jax.experimental.pallas import tpu_sc as plsc

_N = 100000
_D = 128
_EPS = 1e-5

_NW = 32
_PW = 3200
_NPAD = _NW * _PW
_NCH = _PW // 16

_BLK = 20000
_NBLK = _N // _BLK

_WIN0, _SH0, _MNPS0, _STRX0 = 12, 0, 3200, 80
_WIN1, _SH1, _MNPS1, _STRX1 = 24, 6, 882, 42


def _sc_idx_body(b_hbm, y_hbm, x_hbm,
                 bwi0_hbm, cy0_hbm, cx0_hbm, bwi1_hbm, cy1_hbm, cx1_hbm,
                 stats_hbm,
                 bbuf, ybuf, xbuf, ob0, oy0, ox0, ob1, oy1, ox1, stbuf, sem):
    wid = lax.axis_index("s") * 2 + lax.axis_index("c")
    cin = [pltpu.async_copy(b_hbm.at[wid], bbuf, sem),
           pltpu.async_copy(y_hbm.at[wid], ybuf, sem),
           pltpu.async_copy(x_hbm.at[wid], xbuf, sem)]
    for c in cin:
        c.wait()
    lane = lax.iota(jnp.int32, 16)
    base = wid * _PW
    zero = jnp.zeros((16,), jnp.float32)

    def chunk(i, accs):
        off = i * 16
        bb = bbuf[pl.ds(off, 16)]
        yy = ybuf[pl.ds(off, 16)]
        xx = xbuf[pl.ds(off, 16)]
        wx0 = lax.div(xx, _WIN0)
        wy0 = lax.div(yy, _WIN0)
        cx0 = xx - wx0 * _WIN0
        cy0 = yy - wy0 * _WIN0
        ob0[pl.ds(off, 16)] = bb * _MNPS0 + wx0 * _STRX0 + wy0 * 2
        oy0[pl.ds(off, 16)] = cy0
        ox0[pl.ds(off, 16)] = cx0
        sx1 = xx + _SH1
        sy1 = yy + _SH1
        wx1 = lax.div(sx1, _WIN1)
        wy1 = lax.div(sy1, _WIN1)
        cx1 = sx1 - wx1 * _WIN1
        cy1 = sy1 - wy1 * _WIN1
        ob1[pl.ds(off, 16)] = bb * _MNPS1 + wx1 * _STRX1 + wy1 * 2
        oy1[pl.ds(off, 16)] = cy1
        ox1[pl.ds(off, 16)] = cx1
        valid = (base + off + lane) < _N
        x0 = jnp.where(valid, cx0.astype(jnp.float32) - (_WIN0 / 2.0), 0.0)
        y0 = jnp.where(valid, cy0.astype(jnp.float32) - (_WIN0 / 2.0), 0.0)
        x1 = jnp.where(valid, cx1.astype(jnp.float32) - (_WIN1 / 2.0), 0.0)
        y1 = jnp.where(valid, cy1.astype(jnp.float32) - (_WIN1 / 2.0), 0.0)
        return (accs[0] + x0, accs[1] + y0, accs[2] + x0 * x0,
                accs[3] + y0 * y0, accs[4] + x0 * y0,
                accs[5] + x1, accs[6] + y1, accs[7] + x1 * x1,
                accs[8] + y1 * y1, accs[9] + x1 * y1)

    accs = lax.fori_loop(0, _NCH, chunk, (zero,) * 10, unroll=4)
    for k in range(10):
        stbuf[k] = accs[k]
    cout = [pltpu.async_copy(ob0, bwi0_hbm.at[wid], sem),
            pltpu.async_copy(oy0, cy0_hbm.at[wid], sem),
            pltpu.async_copy(ox0, cx0_hbm.at[wid], sem),
            pltpu.async_copy(ob1, bwi1_hbm.at[wid], sem),
            pltpu.async_copy(oy1, cy1_hbm.at[wid], sem),
            pltpu.async_copy(ox1, cx1_hbm.at[wid], sem),
            pltpu.async_copy(stbuf, stats_hbm.at[wid], sem)]
    for c in cout:
        c.wait()


def _pe_body(cx0_ref, cy0_ref, cx1_ref, cy1_ref, stats_ref,
             w1t0_ref, g0_ref, be0_ref, w2t0_ref, b20_ref,
             w1t1_ref, g1_ref, be1_ref, w2t1_ref, b21_ref,
             pe0_ref, pe1_ref):
    st = jnp.sum(stats_ref[...], axis=0, keepdims=True)
    inv_n = 1.0 / _N

    def moment(k):
        return jnp.sum(st[0:1, 16 * k:16 * k + 16], axis=1, keepdims=True)

    def stage(cx_ref, cy_ref, half, k0,
              w1t_ref, g_ref, be_ref, w2t_ref, b2_ref, out_ref):
        mx = moment(k0 + 0) * inv_n
        my = moment(k0 + 1) * inv_n
        vxx = moment(k0 + 2) * inv_n - mx * mx
        vyy = moment(k0 + 3) * inv_n - my * my
        vxy = moment(k0 + 4) * inv_n - mx * my
        w1x = w1t_ref[0:1, :]
        w1y = w1t_ref[1:2, :]
        var = vxx * w1x * w1x + vyy * w1y * w1y + 2.0 * vxy * w1x * w1y
        scale = g_ref[...] * lax.rsqrt(var + _EPS)
        a1 = w1x * scale
        a2 = w1y * scale
        off = be_ref[...] - (mx * a1 + my * a2)
        x = cx_ref[0].astype(jnp.float32) - half
        y = cy_ref[0].astype(jnp.float32) - half
        xy = jnp.concatenate(
            [x, y, jnp.ones((1, _BLK), jnp.float32),
             jnp.zeros((5, _BLK), jnp.float32)], axis=0)
        a8 = jnp.concatenate(
            [a1, a2, off, jnp.zeros((5, _D), jnp.float32)], axis=0)
        h = lax.dot_general(xy.astype(jnp.bfloat16), a8.astype(jnp.bfloat16),
                            (((0,), (0,)), ((), ())),
                            preferred_element_type=jnp.float32)
        h = jnp.maximum(h, 0.0).astype(jnp.bfloat16)
        out_ref[...] = lax.dot_general(
            h, w2t_ref[...], (((1,), (0,)), ((), ())),
            preferred_element_type=jnp.float32) + b2_ref[...]

    stage(cx0_ref, cy0_ref, _WIN0 / 2.0, 0,
          w1t0_ref, g0_ref, be0_ref, w2t0_ref, b20_ref, pe0_ref)
    stage(cx1_ref, cy1_ref, _WIN1 / 2.0, 5,
          w1t1_ref, g1_ref, be1_ref, w2t1_ref, b21_ref, pe1_ref)


def kernel(box_features, box_coords, w1_0, b1_0, gamma_0, beta_0, w2_0, b2_0,
           w1_1, b1_1, gamma_1, beta_1, w2_1, b2_1):
    coors = box_coords.astype(jnp.int32)
    pad = (0, _NPAD - _N)
    b2d = jnp.pad(coors[:, 0], pad).reshape(_NW, _PW)
    y2d = jnp.pad(coors[:, 2], pad).reshape(_NW, _PW)
    x2d = jnp.pad(coors[:, 3], pad).reshape(_NW, _PW)

    i2d = jax.ShapeDtypeStruct((_NW, _PW), jnp.int32)
    mesh = plsc.VectorSubcoreMesh(core_axis_name="c", subcore_axis_name="s")
    sc_call = pl.kernel(
        _sc_idx_body,
        out_type=[i2d, i2d, i2d, i2d, i2d, i2d,
                  jax.ShapeDtypeStruct((_NW, 10, 16), jnp.float32)],
        mesh=mesh,
        scratch_types=[pltpu.VMEM((_PW,), jnp.int32)] * 9
                      + [pltpu.VMEM((10, 16), jnp.float32),
                         pltpu.SemaphoreType.DMA],
    )
    bwi0_2d, cy0_2d, cx0_2d, bwi1_2d, cy1_2d, cx1_2d, stats = sc_call(
        b2d, y2d, x2d)

    cy0f = cy0_2d.reshape(_NPAD)[:_N]
    cx0f = cx0_2d.reshape(_NPAD)[:_N]
    cy1f = cy1_2d.reshape(_NPAD)[:_N]
    cx1f = cx1_2d.reshape(_NPAD)[:_N]

    cx0c = cx0f.reshape(_NBLK, 1, _BLK)
    cy0c = cy0f.reshape(_NBLK, 1, _BLK)
    cx1c = cx1f.reshape(_NBLK, 1, _BLK)
    cy1c = cy1f.reshape(_NBLK, 1, _BLK)

    col_spec = pl.BlockSpec((1, 1, _BLK), lambda i: (i, 0, 0))
    full = lambda shape: pl.BlockSpec(shape, lambda i: (0,) * len(shape))
    pe_spec = pl.BlockSpec((_BLK, _D), lambda i: (i, 0))
    peshape = jax.ShapeDtypeStruct((_N, _D), jnp.float32)

    pe0, pe1 = pl.pallas_call(
        _pe_body,
        grid=(_NBLK,),
        in_specs=[col_spec, col_spec, col_spec, col_spec,
                  full((_NW, 160)),
                  full((2, _D)), full((1, _D)), full((1, _D)),
                  full((_D, _D)), full((1, _D)),
                  full((2, _D)), full((1, _D)), full((1, _D)),
                  full((_D, _D)), full((1, _D))],
        out_specs=(pe_spec, pe_spec),
        out_shape=(peshape, peshape),
    )(cx0c, cy0c, cx1c, cy1c, stats.reshape(_NW, 160),
      w1_0.T, gamma_0.reshape(1, _D), beta_0.reshape(1, _D),
      w2_0.T.astype(jnp.bfloat16), b2_0.reshape(1, _D),
      w1_1.T, gamma_1.reshape(1, _D), beta_1.reshape(1, _D),
      w2_1.T.astype(jnp.bfloat16), b2_1.reshape(1, _D))

    bwi0 = bwi0_2d.reshape(_NPAD)[:_N]
    bwi1 = bwi1_2d.reshape(_NPAD)[:_N]
    z = jnp.zeros((_N,), jnp.int32)
    ciw0 = jnp.stack([z, cy0f, cx0f], axis=-1)
    ciw1 = jnp.stack([z, cy1f, cx1f], axis=-1)
    return (box_features, pe0, pe1, bwi0, bwi1, ciw0, ciw1)


# speedup vs baseline: 1.0983x; 1.0983x over previous
import functools

import jax
import jax.numpy as jnp
from jax import lax
from jax.experimental import pallas as pl
from jax.experimental.pallas import tpu as pltpu
from jax.experimental.pallas import tpu_sc as plsc

_N = 100000
_D = 128
_EPS = 1e-5

_NW = 32
_PW = 3200
_NPAD = _NW * _PW
_NCH = _PW // 16

_BLK = 20000
_NBLK = _N // _BLK

_WIN0, _SH0, _MNPS0, _STRX0 = 12, 0, 3200, 80
_WIN1, _SH1, _MNPS1, _STRX1 = 24, 6, 882, 42


def _sc_idx_body(b_hbm, y_hbm, x_hbm,
                 bwi0_hbm, cy0_hbm, cx0_hbm, bwi1_hbm, cy1_hbm, cx1_hbm,
                 stats_hbm,
                 bbuf, ybuf, xbuf, ob0, oy0, ox0, ob1, oy1, ox1, stbuf, sem):
    wid = lax.axis_index("s") * 2 + lax.axis_index("c")
    cin = [pltpu.async_copy(b_hbm.at[wid], bbuf, sem),
           pltpu.async_copy(y_hbm.at[wid], ybuf, sem),
           pltpu.async_copy(x_hbm.at[wid], xbuf, sem)]
    for c in cin:
        c.wait()
    lane = lax.iota(jnp.int32, 16)
    base = wid * _PW
    zero = jnp.zeros((16,), jnp.float32)

    def chunk(i, accs):
        off = i * 16
        bb = bbuf[pl.ds(off, 16)]
        yy = ybuf[pl.ds(off, 16)]
        xx = xbuf[pl.ds(off, 16)]
        wx0 = lax.div(xx, _WIN0)
        wy0 = lax.div(yy, _WIN0)
        cx0 = xx - wx0 * _WIN0
        cy0 = yy - wy0 * _WIN0
        ob0[pl.ds(off, 16)] = bb * _MNPS0 + wx0 * _STRX0 + wy0 * 2
        oy0[pl.ds(off, 16)] = cy0
        ox0[pl.ds(off, 16)] = cx0
        sx1 = xx + _SH1
        sy1 = yy + _SH1
        wx1 = lax.div(sx1, _WIN1)
        wy1 = lax.div(sy1, _WIN1)
        cx1 = sx1 - wx1 * _WIN1
        cy1 = sy1 - wy1 * _WIN1
        ob1[pl.ds(off, 16)] = bb * _MNPS1 + wx1 * _STRX1 + wy1 * 2
        oy1[pl.ds(off, 16)] = cy1
        ox1[pl.ds(off, 16)] = cx1
        valid = (base + off + lane) < _N
        x0 = jnp.where(valid, cx0.astype(jnp.float32) - (_WIN0 / 2.0), 0.0)
        y0 = jnp.where(valid, cy0.astype(jnp.float32) - (_WIN0 / 2.0), 0.0)
        x1 = jnp.where(valid, cx1.astype(jnp.float32) - (_WIN1 / 2.0), 0.0)
        y1 = jnp.where(valid, cy1.astype(jnp.float32) - (_WIN1 / 2.0), 0.0)
        return (accs[0] + x0, accs[1] + y0, accs[2] + x0 * x0,
                accs[3] + y0 * y0, accs[4] + x0 * y0,
                accs[5] + x1, accs[6] + y1, accs[7] + x1 * x1,
                accs[8] + y1 * y1, accs[9] + x1 * y1)

    accs = lax.fori_loop(0, _NCH, chunk, (zero,) * 10)
    for k in range(10):
        stbuf[k] = accs[k]
    cout = [pltpu.async_copy(ob0, bwi0_hbm.at[wid], sem),
            pltpu.async_copy(oy0, cy0_hbm.at[wid], sem),
            pltpu.async_copy(ox0, cx0_hbm.at[wid], sem),
            pltpu.async_copy(ob1, bwi1_hbm.at[wid], sem),
            pltpu.async_copy(oy1, cy1_hbm.at[wid], sem),
            pltpu.async_copy(ox1, cx1_hbm.at[wid], sem),
            pltpu.async_copy(stbuf, stats_hbm.at[wid], sem)]
    for c in cout:
        c.wait()


def _pe_body(cx0_ref, cy0_ref, cx1_ref, cy1_ref, stats_ref,
             w1t0_ref, g0_ref, be0_ref, w2t0_ref, b20_ref,
             w1t1_ref, g1_ref, be1_ref, w2t1_ref, b21_ref,
             pe0_ref, pe1_ref):
    st = jnp.sum(stats_ref[...], axis=0, keepdims=True)
    inv_n = 1.0 / _N

    def moment(k):
        return jnp.sum(st[0:1, 16 * k:16 * k + 16], axis=1, keepdims=True)

    def stage(cx_ref, cy_ref, half, k0,
              w1t_ref, g_ref, be_ref, w2t_ref, b2_ref, out_ref):
        mx = moment(k0 + 0) * inv_n
        my = moment(k0 + 1) * inv_n
        vxx = moment(k0 + 2) * inv_n - mx * mx
        vyy = moment(k0 + 3) * inv_n - my * my
        vxy = moment(k0 + 4) * inv_n - mx * my
        w1x = w1t_ref[0:1, :]
        w1y = w1t_ref[1:2, :]
        var = vxx * w1x * w1x + vyy * w1y * w1y + 2.0 * vxy * w1x * w1y
        scale = g_ref[...] * lax.rsqrt(var + _EPS)
        a1 = w1x * scale
        a2 = w1y * scale
        off = be_ref[...] - (mx * a1 + my * a2)
        x = cx_ref[0].astype(jnp.float32) - half
        y = cy_ref[0].astype(jnp.float32) - half
        xy = jnp.concatenate(
            [x, y, jnp.ones((1, _BLK), jnp.float32),
             jnp.zeros((5, _BLK), jnp.float32)], axis=0)
        a8 = jnp.concatenate(
            [a1, a2, off, jnp.zeros((5, _D), jnp.float32)], axis=0)
        h = lax.dot_general(xy.astype(jnp.bfloat16), a8.astype(jnp.bfloat16),
                            (((0,), (0,)), ((), ())),
                            preferred_element_type=jnp.float32)
        h = jnp.maximum(h, 0.0).astype(jnp.bfloat16)
        out_ref[...] = lax.dot_general(
            h, w2t_ref[...], (((1,), (0,)), ((), ())),
            preferred_element_type=jnp.float32) + b2_ref[...]

    stage(cx0_ref, cy0_ref, _WIN0 / 2.0, 0,
          w1t0_ref, g0_ref, be0_ref, w2t0_ref, b20_ref, pe0_ref)
    stage(cx1_ref, cy1_ref, _WIN1 / 2.0, 5,
          w1t1_ref, g1_ref, be1_ref, w2t1_ref, b21_ref, pe1_ref)


def kernel(box_features, box_coords, w1_0, b1_0, gamma_0, beta_0, w2_0, b2_0,
           w1_1, b1_1, gamma_1, beta_1, w2_1, b2_1):
    coors = box_coords.astype(jnp.int32)
    pad = (0, _NPAD - _N)
    b2d = jnp.pad(coors[:, 0], pad).reshape(_NW, _PW)
    y2d = jnp.pad(coors[:, 2], pad).reshape(_NW, _PW)
    x2d = jnp.pad(coors[:, 3], pad).reshape(_NW, _PW)

    i2d = jax.ShapeDtypeStruct((_NW, _PW), jnp.int32)
    mesh = plsc.VectorSubcoreMesh(core_axis_name="c", subcore_axis_name="s")
    sc_call = pl.kernel(
        _sc_idx_body,
        out_type=[i2d, i2d, i2d, i2d, i2d, i2d,
                  jax.ShapeDtypeStruct((_NW, 10, 16), jnp.float32)],
        mesh=mesh,
        scratch_types=[pltpu.VMEM((_PW,), jnp.int32)] * 9
                      + [pltpu.VMEM((10, 16), jnp.float32),
                         pltpu.SemaphoreType.DMA],
    )
    bwi0_2d, cy0_2d, cx0_2d, bwi1_2d, cy1_2d, cx1_2d, stats = sc_call(
        b2d, y2d, x2d)

    cy0f = cy0_2d.reshape(_NPAD)[:_N]
    cx0f = cx0_2d.reshape(_NPAD)[:_N]
    cy1f = cy1_2d.reshape(_NPAD)[:_N]
    cx1f = cx1_2d.reshape(_NPAD)[:_N]

    cx0c = cx0f.reshape(_NBLK, 1, _BLK)
    cy0c = cy0f.reshape(_NBLK, 1, _BLK)
    cx1c = cx1f.reshape(_NBLK, 1, _BLK)
    cy1c = cy1f.reshape(_NBLK, 1, _BLK)

    col_spec = pl.BlockSpec((1, 1, _BLK), lambda i: (i, 0, 0))
    full = lambda shape: pl.BlockSpec(shape, lambda i: (0,) * len(shape))
    pe_spec = pl.BlockSpec((_BLK, _D), lambda i: (i, 0))
    peshape = jax.ShapeDtypeStruct((_N, _D), jnp.float32)

    pe0, pe1 = pl.pallas_call(
        _pe_body,
        grid=(_NBLK,),
        in_specs=[col_spec, col_spec, col_spec, col_spec,
                  full((_NW, 160)),
                  full((2, _D)), full((1, _D)), full((1, _D)),
                  full((_D, _D)), full((1, _D)),
                  full((2, _D)), full((1, _D)), full((1, _D)),
                  full((_D, _D)), full((1, _D))],
        out_specs=(pe_spec, pe_spec),
        out_shape=(peshape, peshape),
    )(cx0c, cy0c, cx1c, cy1c, stats.reshape(_NW, 160),
      w1_0.T, gamma_0.reshape(1, _D), beta_0.reshape(1, _D),
      w2_0.T.astype(jnp.bfloat16), b2_0.reshape(1, _D),
      w1_1.T, gamma_1.reshape(1, _D), beta_1.reshape(1, _D),
      w2_1.T.astype(jnp.bfloat16), b2_1.reshape(1, _D))

    bwi0 = bwi0_2d.reshape(_NPAD)[:_N]
    bwi1 = bwi1_2d.reshape(_NPAD)[:_N]
    z = jnp.zeros((_N,), jnp.int32)
    ciw0 = jnp.stack([z, cy0f, cx0f], axis=-1)
    ciw1 = jnp.stack([z, cy1f, cx1f], axis=-1)
    return (box_features, pe0, pe1, bwi0, bwi1, ciw0, ciw1)


# speedup vs baseline: 1.4177x; 1.2907x over previous
import jax
import jax.numpy as jnp
from jax import lax
from jax.experimental import pallas as pl
from jax.experimental.pallas import tpu as pltpu
from jax.experimental.pallas import tpu_sc as plsc

_N = 100000
_D = 128
_EPS = 1e-5

_NW = 32
_PW = 3200
_NPAD = _NW * _PW
_NCH = _PW // 16

_SR = 8
_SC = _N // _SR
_BLK = 20000
_NBLK = _N // _BLK

_WIN0, _SH0, _MNPS0, _STRX0 = 12, 0, 3200, 80
_WIN1, _SH1, _MNPS1, _STRX1 = 24, 6, 882, 42


def _sc_idx_body(b_hbm, y_hbm, x_hbm,
                 bwi0_hbm, cy0_hbm, cx0_hbm, bwi1_hbm, cy1_hbm, cx1_hbm,
                 bbuf, ybuf, xbuf, ob0, oy0, ox0, ob1, oy1, ox1, sem):
    wid = lax.axis_index("s") * 2 + lax.axis_index("c")
    cin = [pltpu.async_copy(b_hbm.at[wid], bbuf, sem),
           pltpu.async_copy(y_hbm.at[wid], ybuf, sem),
           pltpu.async_copy(x_hbm.at[wid], xbuf, sem)]
    for c in cin:
        c.wait()

    def chunk(i, carry):
        off = i * 16
        bb = bbuf[pl.ds(off, 16)]
        yy = ybuf[pl.ds(off, 16)]
        xx = xbuf[pl.ds(off, 16)]
        wx0 = lax.div(xx, _WIN0)
        wy0 = lax.div(yy, _WIN0)
        ob0[pl.ds(off, 16)] = bb * _MNPS0 + wx0 * _STRX0 + wy0 * 2
        oy0[pl.ds(off, 16)] = yy - wy0 * _WIN0
        ox0[pl.ds(off, 16)] = xx - wx0 * _WIN0
        sx1 = xx + _SH1
        sy1 = yy + _SH1
        wx1 = lax.div(sx1, _WIN1)
        wy1 = lax.div(sy1, _WIN1)
        ob1[pl.ds(off, 16)] = bb * _MNPS1 + wx1 * _STRX1 + wy1 * 2
        oy1[pl.ds(off, 16)] = sy1 - wy1 * _WIN1
        ox1[pl.ds(off, 16)] = sx1 - wx1 * _WIN1
        return carry

    lax.fori_loop(0, _NCH, chunk, 0)
    cout = [pltpu.async_copy(ob0, bwi0_hbm.at[wid], sem),
            pltpu.async_copy(oy0, cy0_hbm.at[wid], sem),
            pltpu.async_copy(ox0, cx0_hbm.at[wid], sem),
            pltpu.async_copy(ob1, bwi1_hbm.at[wid], sem),
            pltpu.async_copy(oy1, cy1_hbm.at[wid], sem),
            pltpu.async_copy(ox1, cx1_hbm.at[wid], sem)]
    for c in cout:
        c.wait()


def _stats_body(y_ref, x_ref, stats_ref):
    yc = y_ref[...]
    xc = x_ref[...]
    wx0 = lax.div(xc, _WIN0)
    wy0 = lax.div(yc, _WIN0)
    x0 = (xc - wx0 * _WIN0).astype(jnp.float32) - (_WIN0 / 2.0)
    y0 = (yc - wy0 * _WIN0).astype(jnp.float32) - (_WIN0 / 2.0)
    sx1 = xc + _SH1
    sy1 = yc + _SH1
    wx1 = lax.div(sx1, _WIN1)
    wy1 = lax.div(sy1, _WIN1)
    x1 = (sx1 - wx1 * _WIN1).astype(jnp.float32) - (_WIN1 / 2.0)
    y1 = (sy1 - wy1 * _WIN1).astype(jnp.float32) - (_WIN1 / 2.0)
    sums = (jnp.sum(x0), jnp.sum(y0), jnp.sum(x0 * x0), jnp.sum(y0 * y0),
            jnp.sum(x0 * y0),
            jnp.sum(x1), jnp.sum(y1), jnp.sum(x1 * x1), jnp.sum(y1 * y1),
            jnp.sum(x1 * y1))
    row = lax.broadcasted_iota(jnp.int32, (8, _D), 0)
    lane = lax.broadcasted_iota(jnp.int32, (8, _D), 1)
    acc = jnp.zeros((8, _D), jnp.float32)
    for k, s in enumerate(sums):
        acc = acc + jnp.where((row == 0) & (lane == k), s, 0.0)
    stats_ref[...] = acc


def _pe_body(xr_ref, yr_ref, stats_ref,
             w1t0_ref, g0_ref, be0_ref, w2t0_ref, b20_ref,
             w1t1_ref, g1_ref, be1_ref, w2t1_ref, b21_ref,
             pe0_ref, pe1_ref):
    stats = stats_ref[...]
    inv_n = 1.0 / _N

    def stage(shift, win, half, k0,
              w1t_ref, g_ref, be_ref, w2t_ref, b2_ref, out_ref):
        sx = stats[0:1, k0 + 0:k0 + 1]
        sy = stats[0:1, k0 + 1:k0 + 2]
        sxx = stats[0:1, k0 + 2:k0 + 3]
        syy = stats[0:1, k0 + 3:k0 + 4]
        sxy = stats[0:1, k0 + 4:k0 + 5]
        mx = sx * inv_n
        my = sy * inv_n
        vxx = sxx * inv_n - mx * mx
        vyy = syy * inv_n - my * my
        vxy = sxy * inv_n - mx * my
        w1x = w1t_ref[0:1, :]
        w1y = w1t_ref[1:2, :]
        var = vxx * w1x * w1x + vyy * w1y * w1y + 2.0 * vxy * w1x * w1y
        scale = g_ref[...] * lax.rsqrt(var + _EPS)
        a1 = w1x * scale
        a2 = w1y * scale
        off = be_ref[...] - (mx * a1 + my * a2)
        xs = xr_ref[0] + shift
        ys = yr_ref[0] + shift
        cx = xs - lax.div(xs, win) * win
        cy = ys - lax.div(ys, win) * win
        x = cx.astype(jnp.float32) - half
        y = cy.astype(jnp.float32) - half
        xy = jnp.concatenate(
            [x, y, jnp.ones((1, _BLK), jnp.float32),
             jnp.zeros((5, _BLK), jnp.float32)], axis=0)
        a8 = jnp.concatenate(
            [a1, a2, off, jnp.zeros((5, _D), jnp.float32)], axis=0)
        h = lax.dot_general(xy.astype(jnp.bfloat16), a8.astype(jnp.bfloat16),
                            (((0,), (0,)), ((), ())),
                            preferred_element_type=jnp.float32)
        h = jnp.maximum(h, 0.0).astype(jnp.bfloat16)
        out_ref[...] = lax.dot_general(
            h, w2t_ref[...], (((1,), (0,)), ((), ())),
            preferred_element_type=jnp.float32) + b2_ref[...]

    stage(_SH0, _WIN0, _WIN0 / 2.0, 0,
          w1t0_ref, g0_ref, be0_ref, w2t0_ref, b20_ref, pe0_ref)
    stage(_SH1, _WIN1, _WIN1 / 2.0, 5,
          w1t1_ref, g1_ref, be1_ref, w2t1_ref, b21_ref, pe1_ref)


def kernel(box_features, box_coords, w1_0, b1_0, gamma_0, beta_0, w2_0, b2_0,
           w1_1, b1_1, gamma_1, beta_1, w2_1, b2_1):
    coors = box_coords.astype(jnp.int32)
    bcol = coors[:, 0]
    ycol = coors[:, 2]
    xcol = coors[:, 3]

    pad = (0, _NPAD - _N)
    b2d = jnp.pad(bcol, pad).reshape(_NW, _PW)
    y2d = jnp.pad(ycol, pad).reshape(_NW, _PW)
    x2d = jnp.pad(xcol, pad).reshape(_NW, _PW)

    i2d = jax.ShapeDtypeStruct((_NW, _PW), jnp.int32)
    mesh = plsc.VectorSubcoreMesh(core_axis_name="c", subcore_axis_name="s")
    sc_call = pl.kernel(
        _sc_idx_body,
        out_type=[i2d, i2d, i2d, i2d, i2d, i2d],
        mesh=mesh,
        scratch_types=[pltpu.VMEM((_PW,), jnp.int32)] * 9
                      + [pltpu.SemaphoreType.DMA],
    )
    bwi0_2d, cy0_2d, cx0_2d, bwi1_2d, cy1_2d, cx1_2d = sc_call(b2d, y2d, x2d)

    stats = pl.pallas_call(
        _stats_body,
        out_shape=jax.ShapeDtypeStruct((8, _D), jnp.float32),
    )(ycol.reshape(_SR, _SC), xcol.reshape(_SR, _SC))

    xrc = xcol.reshape(_NBLK, 1, _BLK)
    yrc = ycol.reshape(_NBLK, 1, _BLK)

    col_spec = pl.BlockSpec((1, 1, _BLK), lambda i: (i, 0, 0))
    full = lambda shape: pl.BlockSpec(shape, lambda i: (0,) * len(shape))
    pe_spec = pl.BlockSpec((_BLK, _D), lambda i: (i, 0))
    peshape = jax.ShapeDtypeStruct((_N, _D), jnp.float32)

    pe0, pe1 = pl.pallas_call(
        _pe_body,
        grid=(_NBLK,),
        in_specs=[col_spec, col_spec,
                  full((8, _D)),
                  full((2, _D)), full((1, _D)), full((1, _D)),
                  full((_D, _D)), full((1, _D)),
                  full((2, _D)), full((1, _D)), full((1, _D)),
                  full((_D, _D)), full((1, _D))],
        out_specs=(pe_spec, pe_spec),
        out_shape=(peshape, peshape),
    )(xrc, yrc, stats,
      w1_0.T, gamma_0.reshape(1, _D), beta_0.reshape(1, _D),
      w2_0.T.astype(jnp.bfloat16), b2_0.reshape(1, _D),
      w1_1.T, gamma_1.reshape(1, _D), beta_1.reshape(1, _D),
      w2_1.T.astype(jnp.bfloat16), b2_1.reshape(1, _D))

    cy0f = cy0_2d.reshape(_NPAD)[:_N]
    cx0f = cx0_2d.reshape(_NPAD)[:_N]
    cy1f = cy1_2d.reshape(_NPAD)[:_N]
    cx1f = cx1_2d.reshape(_NPAD)[:_N]
    bwi0 = bwi0_2d.reshape(_NPAD)[:_N]
    bwi1 = bwi1_2d.reshape(_NPAD)[:_N]
    z = jnp.zeros((_N,), jnp.int32)
    ciw0 = jnp.stack([z, cy0f, cx0f], axis=-1)
    ciw1 = jnp.stack([z, cy1f, cx1f], axis=-1)
    return (box_features, pe0, pe1, bwi0, bwi1, ciw0, ciw1)
